# Initial kernel scaffold; baseline (speedup 1.0000x reference)
import jax
import jax.numpy as jnp
from jax.experimental import pallas as pl


def kernel(x, W_enc, b_enc, threshold, W_dec, b_dec):
    raise NotImplementedError("write your pallas kernel here")


# speedup vs baseline: 9.0089x; 9.0089x over previous
import functools

import jax
import jax.numpy as jnp
from jax import lax
from jax.experimental import pallas as pl
from jax.experimental.pallas import tpu as pltpu

D_IN, H, D_OUT, K = 768, 16384, 768, 64
T = 256
HB = 1024
NHB = H // HB
SEARCH_ITERS = 31
_INF_BITS = 0x7F800000


def _body(x_ref, beff_ref, bdec_ref, we_hbm, wd_hbm, out_ref,
          feat, webuf, wdbuf, wesem, wdsem):
    def we_copy(hb, slot):
        return pltpu.make_async_copy(we_hbm.at[hb], webuf.at[slot], wesem.at[slot])

    def wd_copy(hb, slot):
        return pltpu.make_async_copy(wd_hbm.at[hb], wdbuf.at[slot], wdsem.at[slot])

    we_copy(0, 0).start()
    xt = x_ref[...]
    for hb in range(NHB):
        slot = hb % 2
        if hb + 1 < NHB:
            we_copy(hb + 1, (hb + 1) % 2).start()
        we_copy(hb, slot).wait()
        pre = jnp.dot(xt, webuf[slot], preferred_element_type=jnp.float32)
        pre = pre + beff_ref[0:1, hb * HB:(hb + 1) * HB]
        feat[:, hb * HB:(hb + 1) * HB] = jnp.maximum(pre, 0.0)

    wd_copy(0, 0).start()

    lo0 = jnp.zeros((T, 1), jnp.int32)
    hi0 = jnp.full((T, 1), _INF_BITS, jnp.int32)

    def bs_iter(_, carry):
        lo, hi = carry
        mid = lo + ((hi - lo) >> 1)
        t = lax.bitcast_convert_type(mid, jnp.float32)

        def blk(hb, c):
            fb = feat[:, pl.ds(hb * HB, HB)]
            return c + jnp.sum((fb > t).astype(jnp.float32), axis=1,
                               keepdims=True)

        cnt = lax.fori_loop(0, NHB, blk, jnp.zeros((T, 1), jnp.float32))
        pred = cnt >= float(K)
        return jnp.where(pred, mid, lo), jnp.where(pred, hi, mid)

    lo, _ = lax.fori_loop(0, SEARCH_ITERS, bs_iter, (lo0, hi0))
    thr = lax.bitcast_convert_type(lo, jnp.float32)

    acc = jnp.zeros((T, D_OUT), jnp.float32)
    for hb in range(NHB):
        slot = hb % 2
        if hb + 1 < NHB:
            wd_copy(hb + 1, (hb + 1) % 2).start()
        wd_copy(hb, slot).wait()
        fb = feat[:, hb * HB:(hb + 1) * HB]
        m = jnp.where(fb > thr, fb, 0.0).astype(jnp.bfloat16)
        acc = acc + jnp.dot(m, wdbuf[slot], preferred_element_type=jnp.float32)
    out_ref[...] = acc + bdec_ref[0:1, :]


@jax.jit
def _run(x2, beff, bdec2, we_r, wd_r):
    n_tok = x2.shape[0]
    grid = (n_tok // T,)
    return pl.pallas_call(
        _body,
        grid=grid,
        in_specs=[
            pl.BlockSpec((T, D_IN), lambda i: (i, 0)),
            pl.BlockSpec((1, H), lambda i: (0, 0)),
            pl.BlockSpec((1, D_OUT), lambda i: (0, 0)),
            pl.BlockSpec(memory_space=pltpu.MemorySpace.HBM),
            pl.BlockSpec(memory_space=pltpu.MemorySpace.HBM),
        ],
        out_specs=pl.BlockSpec((T, D_OUT), lambda i: (i, 0)),
        out_shape=jax.ShapeDtypeStruct((n_tok, D_OUT), jnp.float32),
        scratch_shapes=[
            pltpu.VMEM((T, H), jnp.float32),
            pltpu.VMEM((2, D_IN, HB), jnp.float32),
            pltpu.VMEM((2, HB, D_OUT), jnp.bfloat16),
            pltpu.SemaphoreType.DMA((2,)),
            pltpu.SemaphoreType.DMA((2,)),
        ],
        compiler_params=pltpu.CompilerParams(
            dimension_semantics=("arbitrary",),
            vmem_limit_bytes=100 * 2**20,
        ),
    )(x2, beff, bdec2, we_r, wd_r)


def kernel(x, W_enc, b_enc, threshold, W_dec, b_dec):
    B, S, _ = x.shape
    x2 = x.reshape(B * S, D_IN)
    beff = (b_enc - threshold).reshape(1, H)
    bdec2 = b_dec.reshape(1, D_OUT)
    we_r = W_enc.reshape(NHB, HB, D_IN).transpose(0, 2, 1)
    wd_r = W_dec.T.reshape(NHB, HB, D_OUT).astype(jnp.bfloat16)
    out2 = _run(x2, beff, bdec2, we_r, wd_r)
    return out2.reshape(B, S, D_OUT)
